# baseline (speedup 1.0000x reference)
import functools

import jax
import jax.numpy as jnp
from jax import lax
from jax.experimental import pallas as pl
from jax.experimental.pallas import tpu as pltpu
from jax.experimental.pallas import tpu_sc as plsc

_B = 1024
_DIM_IN = 512
_N_ATOMS = 64
_F = 4
_DIM_REST = _DIM_IN - _N_ATOMS * 3
_E = 1024
_MSG_H = 16
_GNN_OUT = _N_ATOMS * _F

_NC = 2
_NS = 16
_NW = _NC * _NS
_LPW = _B // _NW
_L = 16

_BLK = 128

_O_WM1 = 0
_O_BM1 = 112
_O_WM2 = 128
_O_BM2 = 192
_O_WN = 196
_O_BN = 224


def _sqrt_sc(s):
    t = s
    m = jnp.full((_L,), 1.0, jnp.float32)
    for e2 in (16, 8, 4, 2, 1):
        f = float(2 ** e2)
        big = t > f
        t = jnp.where(big, t * (1.0 / f), t)
        m = jnp.where(big, m * float(2.0 ** (e2 / 2)), m)
    for e2 in (16, 8, 4, 2, 1):
        f = float(2 ** e2)
        small = t < (1.0 / f)
        t = jnp.where(small, t * f, t)
        m = jnp.where(small, m * float(2.0 ** (-e2 / 2)), m)
    y = 1.69 - 0.535 * t
    for _ in range(4):
        y = y * (1.5 - 0.5 * t * y * y)
    return m * t * y


def _tanh(z):
    e = jnp.exp(-2.0 * jnp.abs(z))
    return jnp.sign(z) * (1.0 - e) / (1.0 + e)


def _gnn_sc_body(pos_hbm, ei_hbm, wpack_hbm, out_hbm, pos_v, ei_v, h_v, w_v):
    wid = lax.axis_index("s") * _NC + lax.axis_index("c")

    pltpu.sync_copy(pos_hbm.at[wid], pos_v)
    pltpu.sync_copy(ei_hbm, ei_v)
    pltpu.sync_copy(wpack_hbm, w_v)

    wrow = [w_v[r, pl.ds(0, _L)] for r in range(16)]
    sc = lambda k: wrow[k // 16][k % 16]
    w1 = [[sc(_O_WM1 + c * 16 + h) for c in range(7)] for h in range(_MSG_H)]
    b1 = [sc(_O_BM1 + h) for h in range(_MSG_H)]
    w2 = [[sc(_O_WM2 + h * 4 + f) for h in range(_MSG_H)] for f in range(_F)]
    b2 = [sc(_O_BM2 + f) for f in range(_F)]
    wn = [[sc(_O_WN + c * 4 + f) for c in range(7)] for f in range(_F)]
    bn = [sc(_O_BN + f) for f in range(_F)]

    def _zero(r, _):
        for v in range(_LPW // _L):
            h_v[r, pl.ds(v * _L, _L)] = jnp.zeros((_L,), jnp.float32)
        return 0
    lax.fori_loop(0, _GNN_OUT, _zero, 0)

    def _edge(e, _):
        iv = ei_v[e, pl.ds(0, _L)]
        s = iv[0]
        t = iv[1]
        rs = s * 3
        rt = t * 3
        for v in range(_LPW // _L):
            ln = pl.ds(v * _L, _L)
            psx = pos_v[rs, ln]
            psy = pos_v[rs + 1, ln]
            psz = pos_v[rs + 2, ln]
            pdx = pos_v[rt, ln]
            pdy = pos_v[rt + 1, ln]
            pdz = pos_v[rt + 2, ln]
            dx = psx - pdx
            dy = psy - pdy
            dz = psz - pdz
            ss = dx * dx + dy * dy + dz * dz + 1e-8
            dist = _sqrt_sc(ss)
            feat = (psx, psy, psz, pdx, pdy, pdz, dist)
            hid = []
            for h in range(_MSG_H):
                acc = b1[h]
                for c in range(7):
                    acc = acc + w1[h][c] * feat[c]
                hid.append(jnp.maximum(acc, 0.0))
            for f in range(_F):
                mf = b2[f]
                for h in range(_MSG_H):
                    mf = mf + w2[f][h] * hid[h]
                row = t * _F + f
                h_v[row, ln] = h_v[row, ln] + mf
        return 0
    lax.fori_loop(0, _E, _edge, 0)

    def _node(a, _):
        ra = a * 3
        ro = a * _F
        for v in range(_LPW // _L):
            ln = pl.ds(v * _L, _L)
            px = pos_v[ra, ln]
            py = pos_v[ra + 1, ln]
            pz = pos_v[ra + 2, ln]
            g = [h_v[ro + f, ln] for f in range(_F)]
            for f in range(_F):
                z = bn[f] + wn[f][0] * px + wn[f][1] * py + wn[f][2] * pz
                for k in range(_F):
                    z = z + wn[f][3 + k] * g[k]
                h_v[ro + f, ln] = _tanh(z)
        return 0
    lax.fori_loop(0, _N_ATOMS, _node, 0)

    pltpu.sync_copy(h_v, out_hbm.at[wid])


def _dense_tc(xr_ref, g_ref, dw0t_ref, dw0g_ref, db0_ref, dw1_ref, db1_ref,
              dw2_ref, db2_ref, out_ref):
    f32 = jnp.float32
    l0 = jnp.dot(xr_ref[...], dw0t_ref[...], preferred_element_type=f32)
    l0 = l0 + jax.lax.dot_general(g_ref[...], dw0g_ref[...],
                                  (((0,), (0,)), ((), ())),
                                  preferred_element_type=f32)
    h1 = jnp.maximum(l0 + db0_ref[0, :][None, :], 0.0)
    h2 = jnp.maximum(jnp.dot(h1, dw1_ref[...], preferred_element_type=f32)
                     + db1_ref[0, :][None, :], 0.0)
    out_ref[...] = jnp.dot(h2, dw2_ref[...], preferred_element_type=f32) \
        + db2_ref[0, :][None, :]


def kernel(x, edge_index, Wm1, bm1, Wm2, bm2, Wn, bn, Dw0, Db0, Dw1, Db1, Dw2, Db2):
    f32 = jnp.float32
    x_rest = x[:, :_DIM_REST]
    pos_w = x[:, _DIM_REST:].T.reshape(_N_ATOMS * 3, _NW, _LPW).transpose(1, 0, 2)

    wpack = jnp.concatenate([
        Wm1.reshape(-1), bm1, Wm2.reshape(-1), bm2, Wn.reshape(-1), bn,
        jnp.zeros((256 - _O_BN - _F,), f32)]).reshape(16, 16)
    ei_rows = jnp.concatenate(
        [edge_index.T, jnp.zeros((_E, 14), jnp.int32)], axis=1)

    gnn_sc = functools.partial(
        pl.kernel,
        out_type=jax.ShapeDtypeStruct((_NW, _GNN_OUT, _LPW), f32),
        mesh=plsc.VectorSubcoreMesh(core_axis_name="c", subcore_axis_name="s",
                                    num_cores=_NC, num_subcores=_NS),
        compiler_params=pltpu.CompilerParams(use_tc_tiling_on_sc=False),
        scratch_types=[
            pltpu.VMEM((_N_ATOMS * 3, _LPW), f32),
            pltpu.VMEM((_E, _L), jnp.int32),
            pltpu.VMEM((_GNN_OUT, _LPW), f32),
            pltpu.VMEM((16, 16), f32),
        ],
    )(_gnn_sc_body)
    out_w = gnn_sc(pos_w, ei_rows, wpack)
    gnnT = out_w.transpose(1, 0, 2).reshape(_GNN_OUT, _B)

    dw0_top = Dw0[:_DIM_REST]
    dw0_gnn = Dw0[_DIM_REST:]

    grid = (_B // _BLK,)
    full = lambda shape: pl.BlockSpec(shape, lambda i: (0,) * len(shape))
    out = pl.pallas_call(
        _dense_tc,
        grid=grid,
        in_specs=[
            pl.BlockSpec((_BLK, _DIM_REST), lambda i: (i, 0)),
            pl.BlockSpec((_GNN_OUT, _BLK), lambda i: (0, i)),
            full(dw0_top.shape),
            full(dw0_gnn.shape),
            full((1, Db0.shape[0])),
            full(Dw1.shape),
            full((1, Db1.shape[0])),
            full(Dw2.shape),
            full((1, Db2.shape[0])),
        ],
        out_specs=pl.BlockSpec((_BLK, Dw2.shape[1]), lambda i: (i, 0)),
        out_shape=jax.ShapeDtypeStruct((_B, Dw2.shape[1]), f32),
    )(x_rest, gnnT, dw0_top, dw0_gnn, Db0.reshape(1, -1), Dw1,
      Db1.reshape(1, -1), Dw2, Db2.reshape(1, -1))
    return out


# speedup vs baseline: 1.8582x; 1.8582x over previous
import functools

import jax
import jax.numpy as jnp
from jax import lax
from jax.experimental import pallas as pl
from jax.experimental.pallas import tpu as pltpu
from jax.experimental.pallas import tpu_sc as plsc

_B = 1024
_DIM_IN = 512
_N_ATOMS = 64
_F = 4
_DIM_REST = _DIM_IN - _N_ATOMS * 3
_E = 1024
_MSG_H = 16
_GNN_OUT = _N_ATOMS * _F

_NC = 2
_NS = 16
_NW = _NC * _NS
_LPW = _B // _NW
_L = 16

_BLK = 128

_O_WM1 = 0
_O_BM1 = 112
_O_WM2 = 128
_O_BM2 = 192
_O_WN = 196
_O_BN = 224


def _sqrt_sc(s):
    t = s
    m = jnp.full((_L,), 1.0, jnp.float32)
    for e2 in (16, 8, 4, 2, 1):
        f = float(2 ** e2)
        big = t > f
        t = jnp.where(big, t * (1.0 / f), t)
        m = jnp.where(big, m * float(2.0 ** (e2 / 2)), m)
    for e2 in (16, 8, 4, 2, 1):
        f = float(2 ** e2)
        small = t < (1.0 / f)
        t = jnp.where(small, t * f, t)
        m = jnp.where(small, m * float(2.0 ** (-e2 / 2)), m)
    y = 1.69 - 0.535 * t
    for _ in range(4):
        y = y * (1.5 - 0.5 * t * y * y)
    return m * t * y


def _tanh(z):
    e = jnp.exp(-2.0 * jnp.abs(z))
    return jnp.sign(z) * (1.0 - e) / (1.0 + e)


def _gnn_sc_body(pos_hbm, ei_hbm, wpack_hbm, out_hbm, pos_v, ei_v, h_v, w_v,
                 u_v, vv_v):
    wid = lax.axis_index("s") * _NC + lax.axis_index("c")

    pltpu.sync_copy(pos_hbm.at[wid], pos_v)
    pltpu.sync_copy(ei_hbm, ei_v)
    pltpu.sync_copy(wpack_hbm, w_v)

    wrow = [w_v[r, pl.ds(0, _L)] for r in range(16)]
    sc = lambda k: wrow[k // 16][k % 16]
    w1 = [[sc(_O_WM1 + c * 16 + h) for c in range(7)] for h in range(_MSG_H)]
    b1 = [sc(_O_BM1 + h) for h in range(_MSG_H)]
    w2 = [[sc(_O_WM2 + h * 4 + f) for h in range(_MSG_H)] for f in range(_F)]
    b2 = [sc(_O_BM2 + f) for f in range(_F)]
    wn = [[sc(_O_WN + c * 4 + f) for c in range(7)] for f in range(_F)]
    bn = [sc(_O_BN + f) for f in range(_F)]

    def _zero(r, _):
        for v in range(_LPW // _L):
            h_v[r, pl.ds(v * _L, _L)] = jnp.zeros((_L,), jnp.float32)
        return 0
    lax.fori_loop(0, _GNN_OUT, _zero, 0)

    def _pre(a, _):
        ra = a * 3
        ro = a * _MSG_H
        for v in range(_LPW // _L):
            ln = pl.ds(v * _L, _L)
            px = pos_v[ra, ln]
            py = pos_v[ra + 1, ln]
            pz = pos_v[ra + 2, ln]
            for h in range(_MSG_H):
                u_v[ro + h, ln] = (b1[h] + w1[h][0] * px + w1[h][1] * py
                                   + w1[h][2] * pz)
                vv_v[ro + h, ln] = (w1[h][3] * px + w1[h][4] * py
                                    + w1[h][5] * pz)
        return 0
    lax.fori_loop(0, _N_ATOMS, _pre, 0)

    def _edge(e, _):
        iv = ei_v[e, pl.ds(0, _L)]
        s = iv[0]
        t = iv[1]
        rs = s * 3
        rt = t * 3
        us = s * _MSG_H
        ut = t * _MSG_H
        for v in range(_LPW // _L):
            ln = pl.ds(v * _L, _L)
            psx = pos_v[rs, ln]
            psy = pos_v[rs + 1, ln]
            psz = pos_v[rs + 2, ln]
            pdx = pos_v[rt, ln]
            pdy = pos_v[rt + 1, ln]
            pdz = pos_v[rt + 2, ln]
            dx = psx - pdx
            dy = psy - pdy
            dz = psz - pdz
            ss = dx * dx + dy * dy + dz * dz + 1e-8
            dist = _sqrt_sc(ss)
            hid = []
            for h in range(_MSG_H):
                acc = u_v[us + h, ln] + vv_v[ut + h, ln] + w1[h][6] * dist
                hid.append(jnp.maximum(acc, 0.0))
            for f in range(_F):
                mf = b2[f]
                for h in range(_MSG_H):
                    mf = mf + w2[f][h] * hid[h]
                row = t * _F + f
                h_v[row, ln] = h_v[row, ln] + mf
        return 0
    lax.fori_loop(0, _E, _edge, 0)

    def _node(a, _):
        ra = a * 3
        ro = a * _F
        for v in range(_LPW // _L):
            ln = pl.ds(v * _L, _L)
            px = pos_v[ra, ln]
            py = pos_v[ra + 1, ln]
            pz = pos_v[ra + 2, ln]
            g = [h_v[ro + f, ln] for f in range(_F)]
            for f in range(_F):
                z = bn[f] + wn[f][0] * px + wn[f][1] * py + wn[f][2] * pz
                for k in range(_F):
                    z = z + wn[f][3 + k] * g[k]
                h_v[ro + f, ln] = _tanh(z)
        return 0
    lax.fori_loop(0, _N_ATOMS, _node, 0)

    pltpu.sync_copy(h_v, out_hbm.at[wid])


def _dense_tc(xr_ref, g_ref, dw0t_ref, dw0g_ref, db0_ref, dw1_ref, db1_ref,
              dw2_ref, db2_ref, out_ref):
    f32 = jnp.float32
    l0 = jnp.dot(xr_ref[...], dw0t_ref[...], preferred_element_type=f32)
    l0 = l0 + jax.lax.dot_general(g_ref[...], dw0g_ref[...],
                                  (((0,), (0,)), ((), ())),
                                  preferred_element_type=f32)
    h1 = jnp.maximum(l0 + db0_ref[0, :][None, :], 0.0)
    h2 = jnp.maximum(jnp.dot(h1, dw1_ref[...], preferred_element_type=f32)
                     + db1_ref[0, :][None, :], 0.0)
    out_ref[...] = jnp.dot(h2, dw2_ref[...], preferred_element_type=f32) \
        + db2_ref[0, :][None, :]


def kernel(x, edge_index, Wm1, bm1, Wm2, bm2, Wn, bn, Dw0, Db0, Dw1, Db1, Dw2, Db2):
    f32 = jnp.float32
    x_rest = x[:, :_DIM_REST]
    pos_w = x[:, _DIM_REST:].T.reshape(_N_ATOMS * 3, _NW, _LPW).transpose(1, 0, 2)

    wpack = jnp.concatenate([
        Wm1.reshape(-1), bm1, Wm2.reshape(-1), bm2, Wn.reshape(-1), bn,
        jnp.zeros((256 - _O_BN - _F,), f32)]).reshape(16, 16)
    ei_rows = jnp.concatenate(
        [edge_index.T, jnp.zeros((_E, 14), jnp.int32)], axis=1)

    gnn_sc = functools.partial(
        pl.kernel,
        out_type=jax.ShapeDtypeStruct((_NW, _GNN_OUT, _LPW), f32),
        mesh=plsc.VectorSubcoreMesh(core_axis_name="c", subcore_axis_name="s",
                                    num_cores=_NC, num_subcores=_NS),
        compiler_params=pltpu.CompilerParams(use_tc_tiling_on_sc=False),
        scratch_types=[
            pltpu.VMEM((_N_ATOMS * 3, _LPW), f32),
            pltpu.VMEM((_E, _L), jnp.int32),
            pltpu.VMEM((_GNN_OUT, _LPW), f32),
            pltpu.VMEM((16, 16), f32),
            pltpu.VMEM((_N_ATOMS * _MSG_H, _LPW), f32),
            pltpu.VMEM((_N_ATOMS * _MSG_H, _LPW), f32),
        ],
    )(_gnn_sc_body)
    out_w = gnn_sc(pos_w, ei_rows, wpack)
    gnnT = out_w.transpose(1, 0, 2).reshape(_GNN_OUT, _B)

    dw0_top = Dw0[:_DIM_REST]
    dw0_gnn = Dw0[_DIM_REST:]

    grid = (_B // _BLK,)
    full = lambda shape: pl.BlockSpec(shape, lambda i: (0,) * len(shape))
    out = pl.pallas_call(
        _dense_tc,
        grid=grid,
        in_specs=[
            pl.BlockSpec((_BLK, _DIM_REST), lambda i: (i, 0)),
            pl.BlockSpec((_GNN_OUT, _BLK), lambda i: (0, i)),
            full(dw0_top.shape),
            full(dw0_gnn.shape),
            full((1, Db0.shape[0])),
            full(Dw1.shape),
            full((1, Db1.shape[0])),
            full(Dw2.shape),
            full((1, Db2.shape[0])),
        ],
        out_specs=pl.BlockSpec((_BLK, Dw2.shape[1]), lambda i: (i, 0)),
        out_shape=jax.ShapeDtypeStruct((_B, Dw2.shape[1]), f32),
    )(x_rest, gnnT, dw0_top, dw0_gnn, Db0.reshape(1, -1), Dw1,
      Db1.reshape(1, -1), Dw2, Db2.reshape(1, -1))
    return out


# speedup vs baseline: 3.8341x; 2.0633x over previous
import jax
import jax.numpy as jnp
from jax.experimental import pallas as pl

_B = 1024
_DIM_IN = 512
_N_ATOMS = 64
_F = 4
_DIM_REST = _DIM_IN - _N_ATOMS * 3
_E = 1024
_MSG_H = 16

_BLK = 128


def _fused(xr_ref, pos_ref, ei_ref, wm1_ref, bm1_ref, wm2_ref, bm2_ref,
           wn_ref, bn_ref, dw0t_ref, dw0g_ref, db0_ref, dw1_ref, db1_ref,
           dw2_ref, db2_ref, out_ref):
    f32 = jnp.float32
    x_rest = xr_ref[...]
    pos = pos_ref[...]

    src = ei_ref[0, 0, :]
    dst = ei_ref[0, 1, :]
    atoms = jax.lax.broadcasted_iota(jnp.int32, (_E, _N_ATOMS), 1)
    oh_s = (atoms == src[:, None]).astype(f32)
    oh_d = (atoms == dst[:, None]).astype(f32)
    oh_cat = jnp.concatenate([oh_s, oh_d], axis=1)

    wm1 = wm1_ref[...]

    dn_pa = (((0,), (2,)), ((), ()))
    u3 = jax.lax.dot_general(wm1[0:3, :], pos, dn_pa,
                             preferred_element_type=f32)
    v3 = jax.lax.dot_general(wm1[3:6, :], pos, dn_pa,
                             preferred_element_type=f32)
    u3 = u3 + bm1_ref[0, :][:, None, None]
    uv = jnp.concatenate([u3, v3], axis=2)

    pre0 = jax.lax.dot_general(uv, oh_cat, (((2,), (1,)), ((), ())),
                               preferred_element_type=f32)

    diff = jax.lax.dot_general(pos, oh_s - oh_d, (((1,), (1,)), ((), ())),
                               preferred_element_type=f32)
    d = jnp.sqrt(jnp.sum(diff * diff, axis=1) + 1e-8)

    pre = pre0 + wm1[6, :][:, None, None] * d[None, :, :]
    hid = jnp.maximum(pre, 0.0)

    m = jax.lax.dot_general(wm2_ref[...], hid, (((0,), (0,)), ((), ())),
                            preferred_element_type=f32)
    m = m + bm2_ref[0, :][:, None, None]

    agg = jax.lax.dot_general(m, oh_d, (((2,), (0,)), ((), ())),
                              preferred_element_type=f32)

    wn = wn_ref[...]
    dn_l = (((2,), (0,)), ((), ()))
    dn_c = (((1,), (0,)), ((), ()))
    n1 = jax.lax.dot_general(wn[0:3, :], pos, dn_pa,
                             preferred_element_type=f32)
    n2 = jax.lax.dot_general(wn[3:7, :], agg, (((0,), (0,)), ((), ())),
                             preferred_element_type=f32)
    h = jnp.tanh(n1 + n2 + bn_ref[0, :][:, None, None])

    l0g = jax.lax.dot_general(h, dw0g_ref[...],
                              (((2,), (1,)), ((0,), (0,))),
                              preferred_element_type=f32)
    l0 = jnp.dot(x_rest, dw0t_ref[...], preferred_element_type=f32) \
        + l0g[0] + l0g[1] + l0g[2] + l0g[3] + db0_ref[0, :][None, :]
    h1 = jnp.maximum(l0, 0.0)
    h2 = jnp.maximum(jnp.dot(h1, dw1_ref[...], preferred_element_type=f32)
                     + db1_ref[0, :][None, :], 0.0)
    out_ref[...] = jnp.dot(h2, dw2_ref[...], preferred_element_type=f32) \
        + db2_ref[0, :][None, :]


def kernel(x, edge_index, Wm1, bm1, Wm2, bm2, Wn, bn, Dw0, Db0, Dw1, Db1, Dw2, Db2):
    f32 = jnp.float32
    x_rest = x[:, :_DIM_REST]
    pos = x[:, _DIM_REST:].reshape(_B, _N_ATOMS, 3)
    ei3 = edge_index.reshape(1, 2, _E)
    wm1p = jnp.concatenate([Wm1, jnp.zeros((1, _MSG_H), f32)], axis=0)
    wnp = jnp.concatenate([Wn, jnp.zeros((1, _F), f32)], axis=0)
    dw0_top = Dw0[:_DIM_REST]
    dw0_gnn = Dw0[_DIM_REST:].reshape(_N_ATOMS, _F, -1).transpose(1, 0, 2)

    grid = (_B // _BLK,)
    full = lambda shape: pl.BlockSpec(shape, lambda i: (0,) * len(shape))
    out = pl.pallas_call(
        _fused,
        grid=grid,
        in_specs=[
            pl.BlockSpec((_BLK, _DIM_REST), lambda i: (i, 0)),
            pl.BlockSpec((_BLK, _N_ATOMS, 3), lambda i: (i, 0, 0)),
            full((1, 2, _E)),
            full((8, _MSG_H)),
            full((1, _MSG_H)),
            full((_MSG_H, _F)),
            full((1, _F)),
            full((8, _F)),
            full((1, _F)),
            full(dw0_top.shape),
            full(dw0_gnn.shape),
            full((1, Db0.shape[0])),
            full(Dw1.shape),
            full((1, Db1.shape[0])),
            full(Dw2.shape),
            full((1, Db2.shape[0])),
        ],
        out_specs=pl.BlockSpec((_BLK, Dw2.shape[1]), lambda i: (i, 0)),
        out_shape=jax.ShapeDtypeStruct((_B, Dw2.shape[1]), f32),
    )(x_rest, pos, ei3, wm1p, bm1.reshape(1, -1), Wm2, bm2.reshape(1, -1),
      wnp, bn.reshape(1, -1), dw0_top, dw0_gnn, Db0.reshape(1, -1), Dw1,
      Db1.reshape(1, -1), Dw2, Db2.reshape(1, -1))
    return out
